# Initial kernel scaffold; baseline (speedup 1.0000x reference)
import jax
import jax.numpy as jnp
from jax.experimental import pallas as pl


def kernel(x, points, bounds, left_slopes, right_slopes):
    raise NotImplementedError("write your pallas kernel here")


# speedup vs baseline: 1333.0166x; 1333.0166x over previous
import functools

import jax
import jax.numpy as jnp
from jax import lax
from jax.experimental import pallas as pl
from jax.experimental.pallas import tpu as pltpu
from jax.experimental.pallas import tpu_sc as plsc

NUM_CORES = 2
NUM_SUBCORES = 16
NUM_WORKERS = NUM_CORES * NUM_SUBCORES
LANES = 16

B, C, H, W = 2, 96, 384, 384
N_POINTS = 7
SLAB = H * W
NUM_SLABS = B * C
SLABS_PER_W = NUM_SLABS // NUM_WORKERS
CHUNK = 18432
CHUNKS_PER_SLAB = SLAB // CHUNK

_mesh = plsc.VectorSubcoreMesh(
    core_axis_name="c", subcore_axis_name="s",
    num_cores=NUM_CORES, num_subcores=NUM_SUBCORES)


def _bcast(pvec_ref, lane):
    return plsc.load_gather(pvec_ref, [jnp.full((LANES,), lane, jnp.int32)])


@functools.partial(
    pl.kernel,
    out_type=jax.ShapeDtypeStruct((NUM_SLABS, SLAB), jnp.float32),
    mesh=_mesh,
    scratch_types=[
        pltpu.VMEM((CHUNK,), jnp.float32),
        pltpu.VMEM((CHUNK,), jnp.float32),
        pltpu.VMEM((CHUNK,), jnp.float32),
        pltpu.VMEM((CHUNK,), jnp.float32),
        pltpu.VMEM((LANES,), jnp.float32),
        pltpu.VMEM((LANES,), jnp.float32),
        pltpu.VMEM((LANES,), jnp.float32),
        pltpu.SemaphoreType.DMA,
        pltpu.SemaphoreType.DMA,
        pltpu.SemaphoreType.DMA,
        pltpu.SemaphoreType.DMA,
    ],
    compiler_params=pltpu.CompilerParams(needs_layout_passes=False),
)
def _pwlu_sc(x_hbm, params_hbm, out_hbm, ib0, ib1, ob0, ob1, pvec, fpr, sr,
             isem0, isem1, osem0, osem1):
    wid = lax.axis_index("s") * NUM_CORES + lax.axis_index("c")
    iv = lax.iota(jnp.int32, LANES)
    ibufs = (ib0, ib1)
    obufs = (ob0, ob1)
    isems = (isem0, isem1)
    osems = (osem0, osem1)

    for j in range(SLABS_PER_W):
        slab = wid * SLABS_PER_W + j
        ch = lax.rem(slab, C)
        pltpu.sync_copy(params_hbm.at[ch], pvec)

        left = _bcast(pvec, 0)
        right = _bcast(pvec, 1)
        ls = _bcast(pvec, 2)
        rs = _bcast(pvec, 3)
        pm = plsc.load_gather(pvec, [jnp.clip(iv - 1, 0, N_POINTS - 1) + 4])
        pn = plsc.load_gather(pvec, [jnp.clip(iv, 0, N_POINTS - 1) + 4])

        rlen = right - left
        s_int = (pn - pm) / rlen
        s_tab = jnp.where(iv == 0, ls, jnp.where(iv >= N_POINTS, rs, s_int))
        fp_tab = jnp.where(iv == 0, pm - ls * rlen, pm)
        sr[...] = s_tab
        fpr[...] = fp_tab

        sim_left = left - rlen
        inv7l = 1.0 / (N_POINTS * rlen)

        def compute(xin, xout):
            @plsc.parallel_loop(0, CHUNK // LANES, unroll=8)
            def _(i):
                off = i * LANES
                xv = xin[pl.ds(off, LANES)]
                xn = (xv - sim_left) * inv7l
                t7 = xn * jnp.float32(N_POINTS)
                tcl = (jnp.minimum(jnp.maximum(xn, 0.0), 1.001)
                       * jnp.float32(N_POINTS))
                r_i = tcl.astype(jnp.int32)
                r_f = r_i.astype(jnp.float32)
                fpv = plsc.load_gather(fpr, [r_i])
                sv = plsc.load_gather(sr, [r_i])
                xout[pl.ds(off, LANES)] = fpv + (t7 - r_f) * sv

        pltpu.async_copy(x_hbm.at[slab, pl.ds(0, CHUNK)], ibufs[0], isems[0])

        def pair_body(k, _):
            for b in range(2):
                g = 2 * k + b
                nb = 1 - b

                @pl.when(g + 1 < CHUNKS_PER_SLAB)
                def _issue_in():
                    pltpu.async_copy(
                        x_hbm.at[slab, pl.ds((g + 1) * CHUNK, CHUNK)],
                        ibufs[nb], isems[nb])

                pltpu.make_async_copy(
                    x_hbm.at[slab, pl.ds(g * CHUNK, CHUNK)],
                    ibufs[b], isems[b]).wait()

                @pl.when(g >= 2)
                def _drain_out():
                    pltpu.make_async_copy(
                        obufs[b],
                        out_hbm.at[slab, pl.ds((g - 2) * CHUNK, CHUNK)],
                        osems[b]).wait()

                compute(ibufs[b], obufs[b])
                pltpu.async_copy(
                    obufs[b], out_hbm.at[slab, pl.ds(g * CHUNK, CHUNK)],
                    osems[b])
            return 0

        lax.fori_loop(0, CHUNKS_PER_SLAB // 2, pair_body, 0)
        for g in (CHUNKS_PER_SLAB - 2, CHUNKS_PER_SLAB - 1):
            b = g % 2
            pltpu.make_async_copy(
                obufs[b], out_hbm.at[slab, pl.ds(g * CHUNK, CHUNK)],
                osems[b]).wait()


def kernel(x, points, bounds, left_slopes, right_slopes):
    x2 = x.reshape(NUM_SLABS, SLAB)
    pad = jnp.zeros((C, LANES - 4 - N_POINTS), jnp.float32)
    params = jnp.concatenate(
        [bounds, left_slopes[:, None], right_slopes[:, None], points, pad],
        axis=1)
    out = _pwlu_sc(x2, params)
    return out.reshape(x.shape)
